# Initial kernel scaffold; baseline (speedup 1.0000x reference)
import jax
import jax.numpy as jnp
from jax.experimental import pallas as pl


def kernel(h, edge_index, etypes, basis_q, w_comp_q, bias_q, basis_k, w_comp_k, bias_k, basis_v, w_comp_v, bias_v, gn1_w, gn1_b, gn1_ms, gn2_w, gn2_b, gn2_ms, o_w, o_b, ffn1_w, ffn1_b, ffn2_w, ffn2_b):
    raise NotImplementedError("write your pallas kernel here")


# speedup vs baseline: 19.0623x; 19.0623x over previous
import functools
import jax
import jax.numpy as jnp
from jax import lax
from jax.experimental import pallas as pl
from jax.experimental.pallas import tpu as pltpu
from jax.experimental.pallas import tpu_sc as plsc

N = 10000
E = 320000
D = 128
H = 8
DH = 16
R = 9

NC = 2
NS = 16
NW = NC * NS
EPT = E // NW
CH = 128
NFULL = EPT // CH
TAIL = EPT - NFULL * CH
NCHUNK = NFULL + 1
AGG_ROWS = N + 16
RPT = 624

_mesh = functools.partial(
    plsc.VectorSubcoreMesh,
    core_axis_name="c", subcore_axis_name="s",
    num_cores=NC, num_subcores=NS)


def _gn_body(t_ref, w_ref, b_ref, ms_ref, o_ref):
    t = t_ref[...]
    mean = jnp.mean(t, axis=0)
    sub = t - mean * ms_ref[...]
    std = jnp.sqrt(jnp.mean(sub * sub, axis=0) + 1e-06)
    o_ref[...] = w_ref[...] * sub / std + b_ref[...]


def _graph_norm_tc(t, w, b, ms):
    return pl.pallas_call(
        _gn_body,
        out_shape=jax.ShapeDtypeStruct((N, D), jnp.float32),
    )(t, w, b, ms)


def _tables_body(hn_ref, bq_ref, wq_ref, bk_ref, wk_ref, bv_ref, wv_ref,
                 oq_ref, ok_ref, ov_ref):
    hn = hn_ref[...]
    for b_ref, w_ref, o_ref in ((bq_ref, wq_ref, oq_ref),
                                (bk_ref, wk_ref, ok_ref),
                                (bv_ref, wv_ref, ov_ref)):
        W = w_ref[0, 0, 0] * b_ref[0]
        for bb in range(1, R):
            W = W + w_ref[0, 0, bb] * b_ref[bb]
        o_ref[0] = jnp.dot(hn, W, preferred_element_type=jnp.float32,
                 precision=lax.Precision.HIGHEST)


def _tables_tc(hn, basis_q, w_comp_q, basis_k, w_comp_k, basis_v, w_comp_v):
    BR = 2000
    full3 = pl.BlockSpec((R, D, D), lambda r, i: (0, 0, 0))
    hblk = pl.BlockSpec((BR, D), lambda r, i: (i, 0))
    wrow = pl.BlockSpec((1, 1, R), lambda r, i: (r, 0, 0))
    outb = pl.BlockSpec((1, BR, D), lambda r, i: (r, i, 0))
    out = jax.ShapeDtypeStruct((R, N, D), jnp.float32)
    return pl.pallas_call(
        _tables_body,
        grid=(R, N // BR),
        in_specs=[hblk, full3, wrow, full3, wrow, full3, wrow],
        out_specs=[outb, outb, outb],
        out_shape=[out, out, out],
    )(hn, basis_q, w_comp_q.reshape(R, 1, R), basis_k,
      w_comp_k.reshape(R, 1, R), basis_v, w_comp_v.reshape(R, 1, R))


def _build_indices(src_h, et_h, dst_h, base, tmp_s, tmp_e, tmp_d,
                   gidx2, didxg2, didxs2):
    def load(j, n):
        pltpu.sync_copy(src_h.at[pl.ds(base + j * CH, n)], tmp_s.at[pl.ds(0, n)])
        pltpu.sync_copy(dst_h.at[pl.ds(base + j * CH, n)], tmp_d.at[pl.ds(0, n)])
        if et_h is not None:
            pltpu.sync_copy(et_h.at[pl.ds(base + j * CH, n)], tmp_e.at[pl.ds(0, n)])

    def emit(j, kmax):
        for k in range(8):
            sl = pl.ds(k * 16, 16)
            if k < kmax:
                if gidx2 is not None:
                    gidx2[j, sl] = tmp_e[sl] * N + tmp_s[sl]
                if didxg2 is not None:
                    didxg2[j, sl] = tmp_d[sl]
                if didxs2 is not None:
                    didxs2[j, sl] = tmp_d[sl]
            else:
                if gidx2 is not None:
                    gidx2[j, sl] = jnp.zeros((16,), jnp.int32)
                if didxg2 is not None:
                    didxg2[j, sl] = jnp.zeros((16,), jnp.int32)
                if didxs2 is not None:
                    didxs2[j, sl] = jnp.full((16,), N, jnp.int32)

    def body(j, carry):
        load(j, CH)
        emit(j, 8)
        return carry

    lax.fori_loop(0, NFULL, body, 0)
    load(NFULL, TAIL)
    emit(NFULL, TAIL // 16)


def _phase1_body(tq_h, tk_h, tv_h, src_h, dst_h, et_h, z_h,
                 oq_h, ok_h, ov_h,
                 gidx2, didxs2, tmp_s, tmp_e, tmp_d, zbuf, rbuf, agg):
    cid = lax.axis_index("c")
    sid = lax.axis_index("s")
    wid = sid * NC + cid
    base = wid * EPT

    _build_indices(src_h, et_h, dst_h, base, tmp_s, tmp_e, tmp_d,
                   gidx2, None, didxs2)
    pltpu.sync_copy(z_h, zbuf)

    r0 = sid * RPT
    for t_h, o_h in ((tq_h, oq_h), (tk_h, ok_h), (tv_h, ov_h)):
        def zero_body(k, carry):
            pltpu.sync_copy(zbuf, agg.at[pl.ds(r0 + k * 16, 16)])
            return carry

        lax.fori_loop(0, RPT // 16, zero_body, 0)

        @pl.when(sid == 0)
        def _():
            for k in range((AGG_ROWS - NS * RPT) // 16):
                pltpu.sync_copy(zbuf, agg.at[pl.ds(NS * RPT + k * 16, 16)])
        plsc.subcore_barrier()

        def acc_body(j, carry):
            pltpu.sync_copy(t_h.at[gidx2.at[j]], rbuf)
            pltpu.sync_copy(rbuf, agg.at[didxs2.at[j]], add=True)
            return carry

        lax.fori_loop(0, NCHUNK, acc_body, 0)
        plsc.subcore_barrier()

        for k in range(4):
            pltpu.sync_copy(agg.at[pl.ds(r0 + k * CH, CH)], rbuf)
            pltpu.sync_copy(rbuf, o_h.at[cid, pl.ds(r0 + k * CH, CH)])
        rem = RPT - 4 * CH
        pltpu.sync_copy(agg.at[pl.ds(r0 + 4 * CH, rem)], rbuf.at[pl.ds(0, rem)])
        pltpu.sync_copy(rbuf.at[pl.ds(0, rem)], o_h.at[cid, pl.ds(r0 + 4 * CH, rem)])

        @pl.when(sid == 0)
        def _():
            nrem = N - NS * RPT
            pltpu.sync_copy(agg.at[pl.ds(NS * RPT, nrem)], rbuf.at[pl.ds(0, nrem)])
            pltpu.sync_copy(rbuf.at[pl.ds(0, nrem)], o_h.at[cid, pl.ds(NS * RPT, nrem)])
        plsc.subcore_barrier()


def _phase1_sc(tq, tk, tv, src, dst, etypes, z128):
    out = jax.ShapeDtypeStruct((NC, N, D), jnp.float32)
    return pl.kernel(
        _phase1_body,
        out_type=[out, out, out],
        mesh=_mesh(),
        scratch_types=[
            pltpu.VMEM((NCHUNK, CH), jnp.int32),
            pltpu.VMEM((NCHUNK, CH), jnp.int32),
            pltpu.VMEM((CH,), jnp.int32),
            pltpu.VMEM((CH,), jnp.int32),
            pltpu.VMEM((CH,), jnp.int32),
            pltpu.VMEM((16, D), jnp.float32),
            pltpu.VMEM((CH, D), jnp.float32),
            pltpu.VMEM_SHARED((AGG_ROWS, D), jnp.float32),
        ],
    )(tq, tk, tv, src, dst, etypes, z128)


def _qkv_body(aq_ref, ak_ref, av_ref, bq_ref, bk_ref, bv_ref,
              qt_ref, kt_ref, vt_ref):
    qt_ref[...] = jnp.maximum(aq_ref[0] + aq_ref[1] + bq_ref[...], 0.0)
    kt_ref[...] = jnp.maximum(ak_ref[0] + ak_ref[1] + bk_ref[...], 0.0)
    vt_ref[...] = jnp.maximum(av_ref[0] + av_ref[1] + bv_ref[...], 0.0)


def _qkv_tc(aggq, aggk, aggv, bias_q, bias_k, bias_v):
    BR = 2000
    inb = pl.BlockSpec((NC, BR, D), lambda i: (0, i, 0))
    bb = pl.BlockSpec((D,), lambda i: (0,))
    ob = pl.BlockSpec((BR, D), lambda i: (i, 0))
    os = jax.ShapeDtypeStruct((N, D), jnp.float32)
    return pl.pallas_call(
        _qkv_body,
        grid=(N // BR,),
        in_specs=[inb, inb, inb, bb, bb, bb],
        out_specs=[ob, ob, ob],
        out_shape=[os, os, os],
    )(aggq, aggk, aggv, bias_q, bias_k, bias_v)


C2 = 64
NF2 = EPT // C2
TAIL2 = EPT - NF2 * C2


def _gather_body(kt_h, qt_h, src_h, dst_h, kg_h, qg_h,
                 sidx, didxg, tmp_s, tmp_d, kbuf, qbuf):
    cid = lax.axis_index("c")
    sid = lax.axis_index("s")
    wid = sid * NC + cid
    base = wid * EPT

    def chunk(j, nreal):
        pltpu.sync_copy(src_h.at[pl.ds(base + j * C2, nreal)],
                        tmp_s.at[pl.ds(0, nreal)])
        pltpu.sync_copy(dst_h.at[pl.ds(base + j * C2, nreal)],
                        tmp_d.at[pl.ds(0, nreal)])
        for k in range(C2 // 16):
            sl = pl.ds(k * 16, 16)
            if k < nreal // 16:
                sidx[0, sl] = tmp_s[sl]
                didxg[0, sl] = tmp_d[sl]
            else:
                sidx[0, sl] = jnp.zeros((16,), jnp.int32)
                didxg[0, sl] = jnp.zeros((16,), jnp.int32)
        pltpu.sync_copy(kt_h.at[sidx.at[0]], kbuf)
        pltpu.sync_copy(qt_h.at[didxg.at[0]], qbuf)
        pltpu.sync_copy(kbuf.at[pl.ds(0, nreal)],
                        kg_h.at[pl.ds(base + j * C2, nreal)])
        pltpu.sync_copy(qbuf.at[pl.ds(0, nreal)],
                        qg_h.at[pl.ds(base + j * C2, nreal)])

    def cbody(j, carry):
        chunk(j, C2)
        return carry

    lax.fori_loop(0, NF2, cbody, 0)
    chunk(NF2, TAIL2)


def _gather_sc(kt, qt, src, dst):
    out = jax.ShapeDtypeStruct((E, D), jnp.float32)
    return pl.kernel(
        _gather_body,
        out_type=[out, out],
        mesh=_mesh(),
        scratch_types=[
            pltpu.VMEM((1, C2), jnp.int32),
            pltpu.VMEM((1, C2), jnp.int32),
            pltpu.VMEM((C2,), jnp.int32),
            pltpu.VMEM((C2,), jnp.int32),
            pltpu.VMEM((C2, D), jnp.float32),
            pltpu.VMEM((C2, D), jnp.float32),
        ],
    )(kt, qt, src, dst)


def _score_body(kg_ref, qg_ref, sel_ref, scr_ref):
    s = jnp.dot(kg_ref[...] * qg_ref[...], sel_ref[...],
                preferred_element_type=jnp.float32,
                 precision=lax.Precision.HIGHEST)
    scr_ref[...] = jnp.exp(jnp.clip(s * 0.25, -10.0, 10.0))


def _score_tc(kg, qg):
    BR = 4000
    sel = jnp.where(jnp.arange(D)[:, None] // DH == jnp.arange(16)[None, :],
                    1.0, 0.0).astype(jnp.float32)
    inb = pl.BlockSpec((BR, D), lambda i: (i, 0))
    return pl.pallas_call(
        _score_body,
        grid=(E // BR,),
        in_specs=[inb, inb, pl.BlockSpec((D, 16), lambda i: (0, 0))],
        out_specs=pl.BlockSpec((BR, 16), lambda i: (i, 0)),
        out_shape=jax.ShapeDtypeStruct((E, 16), jnp.float32),
    )(kg, qg, sel)


def _phase2_body(vt_h, scr_h, src_h, dst_h, z_h, wv_h, zo_h,
                 sidx, didxs, tmp_s, tmp_d,
                 zbuf, vbuf, obuf, sbuf, acc):
    cid = lax.axis_index("c")
    sid = lax.axis_index("s")
    wid = sid * NC + cid
    base = wid * EPT
    r0 = sid * RPT

    pltpu.sync_copy(z_h, zbuf)

    def zero_acc():
        def zero_body(k, carry):
            pltpu.sync_copy(zbuf, acc.at[pl.ds(r0 + k * 16, 16)])
            return carry

        lax.fori_loop(0, RPT // 16, zero_body, 0)

        @pl.when(sid == 0)
        def _():
            for k in range((AGG_ROWS - NS * RPT) // 16):
                pltpu.sync_copy(zbuf, acc.at[pl.ds(NS * RPT + k * 16, 16)])

    def writeback(o_h):
        def wb_body(k, carry):
            pltpu.sync_copy(acc.at[pl.ds(r0 + k * C2, C2)], obuf)
            pltpu.sync_copy(obuf, o_h.at[cid, pl.ds(r0 + k * C2, C2)])
            return carry

        lax.fori_loop(0, RPT // C2, wb_body, 0)
        rem = RPT - (RPT // C2) * C2
        pltpu.sync_copy(acc.at[pl.ds(r0 + RPT - rem, rem)],
                        obuf.at[pl.ds(0, rem)])
        pltpu.sync_copy(obuf.at[pl.ds(0, rem)],
                        o_h.at[cid, pl.ds(r0 + RPT - rem, rem)])

        @pl.when(sid == 0)
        def _():
            nrem = N - NS * RPT
            pltpu.sync_copy(acc.at[pl.ds(NS * RPT, nrem)],
                            obuf.at[pl.ds(0, nrem)])
            pltpu.sync_copy(obuf.at[pl.ds(0, nrem)],
                            o_h.at[cid, pl.ds(NS * RPT, nrem)])

    def load_didxs(j, nreal):
        pltpu.sync_copy(dst_h.at[pl.ds(base + j * C2, nreal)],
                        tmp_d.at[pl.ds(0, nreal)])
        for k in range(C2 // 16):
            sl = pl.ds(k * 16, 16)
            if k < nreal // 16:
                didxs[0, sl] = tmp_d[sl]
            else:
                didxs[0, sl] = jnp.full((16,), N, jnp.int32)

    zero_acc()
    plsc.subcore_barrier()

    def chunk_a(j, nreal):
        pltpu.sync_copy(src_h.at[pl.ds(base + j * C2, nreal)],
                        tmp_s.at[pl.ds(0, nreal)])
        load_didxs(j, nreal)
        for k in range(C2 // 16):
            sl = pl.ds(k * 16, 16)
            if k < nreal // 16:
                sidx[0, sl] = tmp_s[sl]
            else:
                sidx[0, sl] = jnp.zeros((16,), jnp.int32)
        pltpu.sync_copy(vt_h.at[sidx.at[0]], vbuf)
        pltpu.sync_copy(scr_h.at[pl.ds(base + j * C2, nreal)],
                        sbuf.at[pl.ds(0, nreal)])

        def edge(e, c2):
            es = sbuf[e, pl.ds(0, 16)]
            for hh in range(H):
                obuf[e, pl.ds(hh * DH, DH)] = (
                    vbuf[e, pl.ds(hh * DH, DH)] * es[hh])
            return c2

        lax.fori_loop(0, C2, edge, 0)
        pltpu.sync_copy(obuf, acc.at[didxs.at[0]], add=True)

    def cbody_a(j, carry):
        chunk_a(j, C2)
        return carry

    lax.fori_loop(0, NF2, cbody_a, 0)
    chunk_a(NF2, TAIL2)
    plsc.subcore_barrier()
    writeback(wv_h)
    plsc.subcore_barrier()

    zero_acc()

    def zrow_init(e, carry):
        for k in range(1, D // 16):
            obuf[e, pl.ds(k * 16, 16)] = jnp.zeros((16,), jnp.float32)
        return carry

    lax.fori_loop(0, C2, zrow_init, 0)
    plsc.subcore_barrier()

    def chunk_b(j, nreal):
        load_didxs(j, nreal)
        pltpu.sync_copy(scr_h.at[pl.ds(base + j * C2, nreal)],
                        sbuf.at[pl.ds(0, nreal)])

        def zrow(e, c2):
            obuf[e, pl.ds(0, 16)] = sbuf[e, pl.ds(0, 16)]
            return c2

        lax.fori_loop(0, C2, zrow, 0)
        pltpu.sync_copy(obuf, acc.at[didxs.at[0]], add=True)

    def cbody_b(j, carry):
        chunk_b(j, C2)
        return carry

    lax.fori_loop(0, NF2, cbody_b, 0)
    chunk_b(NF2, TAIL2)
    plsc.subcore_barrier()
    writeback(zo_h)


def _phase2_sc(vt, scr, src, dst, z128):
    out = jax.ShapeDtypeStruct((NC, N, D), jnp.float32)
    return pl.kernel(
        _phase2_body,
        out_type=[out, out],
        mesh=_mesh(),
        scratch_types=[
            pltpu.VMEM((1, C2), jnp.int32),
            pltpu.VMEM((1, C2), jnp.int32),
            pltpu.VMEM((C2,), jnp.int32),
            pltpu.VMEM((C2,), jnp.int32),
            pltpu.VMEM((16, D), jnp.float32),
            pltpu.VMEM((C2, D), jnp.float32),
            pltpu.VMEM((C2, D), jnp.float32),
            pltpu.VMEM((C2, 16), jnp.float32),
            pltpu.VMEM_SHARED((AGG_ROWS, D), jnp.float32),
        ],
    )(vt, scr, src, dst, z128)


def _final_body(wv_ref, zo_ref, srep_ref, ow_ref, ob_ref, gw_ref, gb_ref,
                gms_ref, f1w_ref, f1b_ref, f2w_ref, f2b_ref, o_ref):
    wv = wv_ref[0] + wv_ref[1]
    z = zo_ref[0] + zo_ref[1]
    zrep = jnp.dot(z, srep_ref[...], preferred_element_type=jnp.float32,
                 precision=lax.Precision.HIGHEST)
    attn = wv / (zrep + 1e-06)
    h2 = jnp.dot(attn, ow_ref[...], preferred_element_type=jnp.float32,
                 precision=lax.Precision.HIGHEST) + ob_ref[...]
    mean = jnp.mean(h2, axis=0)
    sub = h2 - mean * gms_ref[...]
    std = jnp.sqrt(jnp.mean(sub * sub, axis=0) + 1e-06)
    h2n = gw_ref[...] * sub / std + gb_ref[...]
    ff = jnp.maximum(
        jnp.dot(h2n, f1w_ref[...], preferred_element_type=jnp.float32,
                 precision=lax.Precision.HIGHEST)
        + f1b_ref[...], 0.0)
    o_ref[...] = jnp.dot(ff, f2w_ref[...],
                         preferred_element_type=jnp.float32,
                 precision=lax.Precision.HIGHEST) + f2b_ref[...]


def _final_tc(wv2, z2, srep, o_w, o_b, gn2_w, gn2_b, gn2_ms,
              ffn1_w, ffn1_b, ffn2_w, ffn2_b):
    return pl.pallas_call(
        _final_body,
        out_shape=jax.ShapeDtypeStruct((N, D), jnp.float32),
    )(wv2, z2, srep, o_w, o_b, gn2_w, gn2_b, gn2_ms,
      ffn1_w, ffn1_b, ffn2_w, ffn2_b)


@jax.jit
def kernel(h, edge_index, etypes, basis_q, w_comp_q, bias_q, basis_k,
           w_comp_k, bias_k, basis_v, w_comp_v, bias_v, gn1_w, gn1_b, gn1_ms,
           gn2_w, gn2_b, gn2_ms, o_w, o_b, ffn1_w, ffn1_b, ffn2_w, ffn2_b):
    src = edge_index[0]
    dst = edge_index[1]

    hn = _graph_norm_tc(h, gn1_w, gn1_b, gn1_ms)
    tq, tk, tv = _tables_tc(hn, basis_q, w_comp_q, basis_k, w_comp_k,
                            basis_v, w_comp_v)
    tq = tq.reshape(R * N, D)
    tk = tk.reshape(R * N, D)
    tv = tv.reshape(R * N, D)

    z128 = jnp.zeros((16, D), jnp.float32)
    aggq, aggk, aggv = _phase1_sc(tq, tk, tv, src, dst, etypes, z128)

    qt, kt, vt = _qkv_tc(aggq, aggk, aggv, bias_q, bias_k, bias_v)

    kg, qg = _gather_sc(kt, qt, src, dst)
    scr = _score_tc(kg, qg)
    wv2, z2 = _phase2_sc(vt, scr, src, dst, z128)

    rows = jnp.arange(D)[:, None]
    cols = jnp.arange(D)[None, :]
    srep = jnp.where((cols // DH == rows) & (rows < H), 1.0, 0.0)
    srep = srep.astype(jnp.float32)

    return _final_tc(wv2, z2, srep, o_w, o_b, gn2_w, gn2_b, gn2_ms,
                     ffn1_w, ffn1_b, ffn2_w, ffn2_b)


# speedup vs baseline: 21.1157x; 1.1077x over previous
import functools
import jax
import jax.numpy as jnp
from jax import lax
from jax.experimental import pallas as pl
from jax.experimental.pallas import tpu as pltpu
from jax.experimental.pallas import tpu_sc as plsc

N = 10000
E = 320000
D = 128
H = 8
DH = 16
R = 9

NC = 2
NS = 16
NW = NC * NS
EPT = E // NW
CH = 80
NCHUNK = EPT // CH
AGG_ROWS = N + 16
RPT = 624

_mesh = functools.partial(
    plsc.VectorSubcoreMesh,
    core_axis_name="c", subcore_axis_name="s",
    num_cores=NC, num_subcores=NS)


def _gn_body(t_ref, w_ref, b_ref, ms_ref, o_ref):
    t = t_ref[...]
    mean = jnp.mean(t, axis=0)
    sub = t - mean * ms_ref[...]
    std = jnp.sqrt(jnp.mean(sub * sub, axis=0) + 1e-06)
    o_ref[...] = w_ref[...] * sub / std + b_ref[...]


def _graph_norm_tc(t, w, b, ms):
    return pl.pallas_call(
        _gn_body,
        out_shape=jax.ShapeDtypeStruct((N, D), jnp.float32),
    )(t, w, b, ms)


def _tables_body(hn_ref, bq_ref, wq_ref, bk_ref, wk_ref, bv_ref, wv_ref,
                 oq_ref, ok_ref, ov_ref):
    hn = hn_ref[...]
    for b_ref, w_ref, o_ref in ((bq_ref, wq_ref, oq_ref),
                                (bk_ref, wk_ref, ok_ref),
                                (bv_ref, wv_ref, ov_ref)):
        W = w_ref[0, 0, 0] * b_ref[0]
        for bb in range(1, R):
            W = W + w_ref[0, 0, bb] * b_ref[bb]
        o_ref[0] = jnp.dot(hn, W, preferred_element_type=jnp.float32,
                 precision=lax.Precision.HIGHEST)


def _tables_tc(hn, basis_q, w_comp_q, basis_k, w_comp_k, basis_v, w_comp_v):
    BR = 2000
    full3 = pl.BlockSpec((R, D, D), lambda r, i: (0, 0, 0))
    hblk = pl.BlockSpec((BR, D), lambda r, i: (i, 0))
    wrow = pl.BlockSpec((1, 1, R), lambda r, i: (r, 0, 0))
    outb = pl.BlockSpec((1, BR, D), lambda r, i: (r, i, 0))
    out = jax.ShapeDtypeStruct((R, N, D), jnp.float32)
    return pl.pallas_call(
        _tables_body,
        grid=(R, N // BR),
        in_specs=[hblk, full3, wrow, full3, wrow, full3, wrow],
        out_specs=[outb, outb, outb],
        out_shape=[out, out, out],
    )(hn, basis_q, w_comp_q.reshape(R, 1, R), basis_k,
      w_comp_k.reshape(R, 1, R), basis_v, w_comp_v.reshape(R, 1, R))


def _phase1_body(tq_h, tk_h, tv_h, src_h, dst_h, et_h, z_h,
                 oq_h, ok_h, ov_h,
                 gidx2, didxs2, tmp_s, tmp_e, tmp_d, zbuf, rbuf0, rbuf1,
                 gsem0, gsem1, agg):
    cid = lax.axis_index("c")
    sid = lax.axis_index("s")
    wid = sid * NC + cid
    base = wid * EPT

    def build_row(j, slot):
        pltpu.sync_copy(src_h.at[pl.ds(base + j * CH, CH)], tmp_s)
        pltpu.sync_copy(et_h.at[pl.ds(base + j * CH, CH)], tmp_e)
        pltpu.sync_copy(dst_h.at[pl.ds(base + j * CH, CH)], tmp_d)
        for k in range(CH // 16):
            sl = pl.ds(k * 16, 16)
            gidx2[slot, sl] = tmp_e[sl] * N + tmp_s[sl]
            didxs2[slot, sl] = tmp_d[sl]

    pltpu.sync_copy(z_h.at[pl.ds(0, 8)], zbuf)

    r0 = sid * RPT
    for t_h, o_h in ((tq_h, oq_h), (tk_h, ok_h), (tv_h, ov_h)):
        def zero_body(k, carry):
            pltpu.sync_copy(zbuf, agg.at[pl.ds(r0 + k * 8, 8)])
            return carry

        lax.fori_loop(0, RPT // 8, zero_body, 0)

        @pl.when(sid == 0)
        def _():
            for k in range((AGG_ROWS - NS * RPT) // 8):
                pltpu.sync_copy(zbuf, agg.at[pl.ds(NS * RPT + k * 8, 8)])
        plsc.subcore_barrier()

        def g_issue(slot, rb, sem):
            pltpu.async_copy(t_h.at[gidx2.at[slot]], rb, sem)

        def g_wait(rb, sem):
            pltpu.make_async_copy(t_h.at[gidx2.at[0]], rb, sem).wait()

        def s_do(slot, rb):
            pltpu.sync_copy(rb, agg.at[didxs2.at[slot]], add=True)

        build_row(0, 0)
        g_issue(0, rbuf0, gsem0)

        def pair(j2, carry):
            a = 2 * j2

            @pl.when(a + 1 < NCHUNK)
            def _():
                build_row(a + 1, 1)
                g_issue(1, rbuf1, gsem1)
            g_wait(rbuf0, gsem0)
            s_do(0, rbuf0)

            @pl.when(a + 2 < NCHUNK)
            def _():
                build_row(a + 2, 0)
                g_issue(0, rbuf0, gsem0)

            @pl.when(a + 1 < NCHUNK)
            def _():
                g_wait(rbuf1, gsem1)
                s_do(1, rbuf1)
            return carry

        lax.fori_loop(0, (NCHUNK + 1) // 2, pair, 0)
        plsc.subcore_barrier()

        for k in range(RPT // CH):
            pltpu.sync_copy(agg.at[pl.ds(r0 + k * CH, CH)], rbuf0)
            pltpu.sync_copy(rbuf0, o_h.at[cid, pl.ds(r0 + k * CH, CH)])
        rem = RPT - (RPT // CH) * CH
        pltpu.sync_copy(agg.at[pl.ds(r0 + RPT - rem, rem)], rbuf0.at[pl.ds(0, rem)])
        pltpu.sync_copy(rbuf0.at[pl.ds(0, rem)],
                        o_h.at[cid, pl.ds(r0 + RPT - rem, rem)])

        @pl.when(sid == 0)
        def _():
            nrem = N - NS * RPT
            pltpu.sync_copy(agg.at[pl.ds(NS * RPT, nrem)], rbuf0.at[pl.ds(0, nrem)])
            pltpu.sync_copy(rbuf0.at[pl.ds(0, nrem)], o_h.at[cid, pl.ds(NS * RPT, nrem)])
        plsc.subcore_barrier()


def _phase1_sc(tq, tk, tv, src, dst, etypes, z128):
    out = jax.ShapeDtypeStruct((NC, N, D), jnp.float32)
    return pl.kernel(
        _phase1_body,
        out_type=[out, out, out],
        mesh=_mesh(),
        scratch_types=[
            pltpu.VMEM((2, CH), jnp.int32),
            pltpu.VMEM((2, CH), jnp.int32),
            pltpu.VMEM((CH,), jnp.int32),
            pltpu.VMEM((CH,), jnp.int32),
            pltpu.VMEM((CH,), jnp.int32),
            pltpu.VMEM((8, D), jnp.float32),
            pltpu.VMEM((CH, D), jnp.float32),
            pltpu.VMEM((CH, D), jnp.float32),
            pltpu.SemaphoreType.DMA,
            pltpu.SemaphoreType.DMA,
            pltpu.VMEM_SHARED((AGG_ROWS, D), jnp.float32),
        ],
    )(tq, tk, tv, src, dst, etypes, z128)


def _qkv_body(aq_ref, ak_ref, av_ref, bq_ref, bk_ref, bv_ref,
              qt_ref, kt_ref, vt_ref):
    qt_ref[...] = jnp.maximum(aq_ref[0] + aq_ref[1] + bq_ref[...], 0.0)
    kt_ref[...] = jnp.maximum(ak_ref[0] + ak_ref[1] + bk_ref[...], 0.0)
    vt_ref[...] = jnp.maximum(av_ref[0] + av_ref[1] + bv_ref[...], 0.0)


def _qkv_tc(aggq, aggk, aggv, bias_q, bias_k, bias_v):
    BR = 2000
    inb = pl.BlockSpec((NC, BR, D), lambda i: (0, i, 0))
    bb = pl.BlockSpec((D,), lambda i: (0,))
    ob = pl.BlockSpec((BR, D), lambda i: (i, 0))
    os = jax.ShapeDtypeStruct((N, D), jnp.float32)
    return pl.pallas_call(
        _qkv_body,
        grid=(N // BR,),
        in_specs=[inb, inb, inb, bb, bb, bb],
        out_specs=[ob, ob, ob],
        out_shape=[os, os, os],
    )(aggq, aggk, aggv, bias_q, bias_k, bias_v)


C2 = 64
NF2 = EPT // C2
TAIL2 = EPT - NF2 * C2


def _gather_body(kt_h, qt_h, src_h, dst_h, kg_h, qg_h,
                 sidx, didxg, tmp_s, tmp_d, kbuf, qbuf):
    cid = lax.axis_index("c")
    sid = lax.axis_index("s")
    wid = sid * NC + cid
    base = wid * EPT

    def chunk(j, nreal):
        pltpu.sync_copy(src_h.at[pl.ds(base + j * C2, nreal)],
                        tmp_s.at[pl.ds(0, nreal)])
        pltpu.sync_copy(dst_h.at[pl.ds(base + j * C2, nreal)],
                        tmp_d.at[pl.ds(0, nreal)])
        for k in range(C2 // 16):
            sl = pl.ds(k * 16, 16)
            if k < nreal // 16:
                sidx[0, sl] = tmp_s[sl]
                didxg[0, sl] = tmp_d[sl]
            else:
                sidx[0, sl] = jnp.zeros((16,), jnp.int32)
                didxg[0, sl] = jnp.zeros((16,), jnp.int32)
        pltpu.sync_copy(kt_h.at[sidx.at[0]], kbuf)
        pltpu.sync_copy(qt_h.at[didxg.at[0]], qbuf)
        pltpu.sync_copy(kbuf.at[pl.ds(0, nreal)],
                        kg_h.at[pl.ds(base + j * C2, nreal)])
        pltpu.sync_copy(qbuf.at[pl.ds(0, nreal)],
                        qg_h.at[pl.ds(base + j * C2, nreal)])

    def cbody(j, carry):
        chunk(j, C2)
        return carry

    lax.fori_loop(0, NF2, cbody, 0)
    chunk(NF2, TAIL2)


def _gather_sc(kt, qt, src, dst):
    out = jax.ShapeDtypeStruct((E, D), jnp.float32)
    return pl.kernel(
        _gather_body,
        out_type=[out, out],
        mesh=_mesh(),
        scratch_types=[
            pltpu.VMEM((1, C2), jnp.int32),
            pltpu.VMEM((1, C2), jnp.int32),
            pltpu.VMEM((C2,), jnp.int32),
            pltpu.VMEM((C2,), jnp.int32),
            pltpu.VMEM((C2, D), jnp.float32),
            pltpu.VMEM((C2, D), jnp.float32),
        ],
    )(kt, qt, src, dst)


def _score_body(kg_ref, qg_ref, sel_ref, scr_ref):
    s = jnp.dot(kg_ref[...] * qg_ref[...], sel_ref[...],
                preferred_element_type=jnp.float32,
                 precision=lax.Precision.HIGHEST)
    scr_ref[...] = jnp.exp(jnp.clip(s * 0.25, -10.0, 10.0))


def _score_tc(kg, qg):
    BR = 4000
    sel = jnp.where(jnp.arange(D)[:, None] // DH == jnp.arange(16)[None, :],
                    1.0, 0.0).astype(jnp.float32)
    inb = pl.BlockSpec((BR, D), lambda i: (i, 0))
    return pl.pallas_call(
        _score_body,
        grid=(E // BR,),
        in_specs=[inb, inb, pl.BlockSpec((D, 16), lambda i: (0, 0))],
        out_specs=pl.BlockSpec((BR, 16), lambda i: (i, 0)),
        out_shape=jax.ShapeDtypeStruct((E, 16), jnp.float32),
    )(kg, qg, sel)


def _phase2_body(vt_h, scr_h, src_h, dst_h, z_h, wv_h, zo_h,
                 sidx, didxs, tmp_s, tmp_d,
                 zbuf, vbuf, obuf, sbuf, acc):
    cid = lax.axis_index("c")
    sid = lax.axis_index("s")
    wid = sid * NC + cid
    base = wid * EPT
    r0 = sid * RPT

    pltpu.sync_copy(z_h, zbuf)

    def zero_acc():
        def zero_body(k, carry):
            pltpu.sync_copy(zbuf, acc.at[pl.ds(r0 + k * 16, 16)])
            return carry

        lax.fori_loop(0, RPT // 16, zero_body, 0)

        @pl.when(sid == 0)
        def _():
            for k in range((AGG_ROWS - NS * RPT) // 16):
                pltpu.sync_copy(zbuf, acc.at[pl.ds(NS * RPT + k * 16, 16)])

    def writeback(o_h):
        def wb_body(k, carry):
            pltpu.sync_copy(acc.at[pl.ds(r0 + k * C2, C2)], obuf)
            pltpu.sync_copy(obuf, o_h.at[cid, pl.ds(r0 + k * C2, C2)])
            return carry

        lax.fori_loop(0, RPT // C2, wb_body, 0)
        rem = RPT - (RPT // C2) * C2
        pltpu.sync_copy(acc.at[pl.ds(r0 + RPT - rem, rem)],
                        obuf.at[pl.ds(0, rem)])
        pltpu.sync_copy(obuf.at[pl.ds(0, rem)],
                        o_h.at[cid, pl.ds(r0 + RPT - rem, rem)])

        @pl.when(sid == 0)
        def _():
            nrem = N - NS * RPT
            pltpu.sync_copy(acc.at[pl.ds(NS * RPT, nrem)],
                            obuf.at[pl.ds(0, nrem)])
            pltpu.sync_copy(obuf.at[pl.ds(0, nrem)],
                            o_h.at[cid, pl.ds(NS * RPT, nrem)])

    def load_didxs(j, nreal):
        pltpu.sync_copy(dst_h.at[pl.ds(base + j * C2, nreal)],
                        tmp_d.at[pl.ds(0, nreal)])
        for k in range(C2 // 16):
            sl = pl.ds(k * 16, 16)
            if k < nreal // 16:
                didxs[0, sl] = tmp_d[sl]
            else:
                didxs[0, sl] = jnp.full((16,), N, jnp.int32)

    zero_acc()
    plsc.subcore_barrier()

    def chunk_a(j, nreal):
        pltpu.sync_copy(src_h.at[pl.ds(base + j * C2, nreal)],
                        tmp_s.at[pl.ds(0, nreal)])
        load_didxs(j, nreal)
        for k in range(C2 // 16):
            sl = pl.ds(k * 16, 16)
            if k < nreal // 16:
                sidx[0, sl] = tmp_s[sl]
            else:
                sidx[0, sl] = jnp.zeros((16,), jnp.int32)
        pltpu.sync_copy(vt_h.at[sidx.at[0]], vbuf)
        pltpu.sync_copy(scr_h.at[pl.ds(base + j * C2, nreal)],
                        sbuf.at[pl.ds(0, nreal)])

        def edge(e, c2):
            es = sbuf[e, pl.ds(0, 16)]
            for hh in range(H):
                obuf[e, pl.ds(hh * DH, DH)] = (
                    vbuf[e, pl.ds(hh * DH, DH)] * es[hh])
            return c2

        lax.fori_loop(0, C2, edge, 0)
        pltpu.sync_copy(obuf, acc.at[didxs.at[0]], add=True)

    def cbody_a(j, carry):
        chunk_a(j, C2)
        return carry

    lax.fori_loop(0, NF2, cbody_a, 0)
    chunk_a(NF2, TAIL2)
    plsc.subcore_barrier()
    writeback(wv_h)
    plsc.subcore_barrier()

    zero_acc()

    def zrow_init(e, carry):
        for k in range(1, D // 16):
            obuf[e, pl.ds(k * 16, 16)] = jnp.zeros((16,), jnp.float32)
        return carry

    lax.fori_loop(0, C2, zrow_init, 0)
    plsc.subcore_barrier()

    def chunk_b(j, nreal):
        load_didxs(j, nreal)
        pltpu.sync_copy(scr_h.at[pl.ds(base + j * C2, nreal)],
                        sbuf.at[pl.ds(0, nreal)])

        def zrow(e, c2):
            obuf[e, pl.ds(0, 16)] = sbuf[e, pl.ds(0, 16)]
            return c2

        lax.fori_loop(0, C2, zrow, 0)
        pltpu.sync_copy(obuf, acc.at[didxs.at[0]], add=True)

    def cbody_b(j, carry):
        chunk_b(j, C2)
        return carry

    lax.fori_loop(0, NF2, cbody_b, 0)
    chunk_b(NF2, TAIL2)
    plsc.subcore_barrier()
    writeback(zo_h)


def _phase2_sc(vt, scr, src, dst, z128):
    out = jax.ShapeDtypeStruct((NC, N, D), jnp.float32)
    return pl.kernel(
        _phase2_body,
        out_type=[out, out],
        mesh=_mesh(),
        scratch_types=[
            pltpu.VMEM((1, C2), jnp.int32),
            pltpu.VMEM((1, C2), jnp.int32),
            pltpu.VMEM((C2,), jnp.int32),
            pltpu.VMEM((C2,), jnp.int32),
            pltpu.VMEM((16, D), jnp.float32),
            pltpu.VMEM((C2, D), jnp.float32),
            pltpu.VMEM((C2, D), jnp.float32),
            pltpu.VMEM((C2, 16), jnp.float32),
            pltpu.VMEM_SHARED((AGG_ROWS, D), jnp.float32),
        ],
    )(vt, scr, src, dst, z128)


def _final_body(wv_ref, zo_ref, srep_ref, ow_ref, ob_ref, gw_ref, gb_ref,
                gms_ref, f1w_ref, f1b_ref, f2w_ref, f2b_ref, o_ref):
    wv = wv_ref[0] + wv_ref[1]
    z = zo_ref[0] + zo_ref[1]
    zrep = jnp.dot(z, srep_ref[...], preferred_element_type=jnp.float32,
                 precision=lax.Precision.HIGHEST)
    attn = wv / (zrep + 1e-06)
    h2 = jnp.dot(attn, ow_ref[...], preferred_element_type=jnp.float32,
                 precision=lax.Precision.HIGHEST) + ob_ref[...]
    mean = jnp.mean(h2, axis=0)
    sub = h2 - mean * gms_ref[...]
    std = jnp.sqrt(jnp.mean(sub * sub, axis=0) + 1e-06)
    h2n = gw_ref[...] * sub / std + gb_ref[...]
    ff = jnp.maximum(
        jnp.dot(h2n, f1w_ref[...], preferred_element_type=jnp.float32,
                 precision=lax.Precision.HIGHEST)
        + f1b_ref[...], 0.0)
    o_ref[...] = jnp.dot(ff, f2w_ref[...],
                         preferred_element_type=jnp.float32,
                 precision=lax.Precision.HIGHEST) + f2b_ref[...]


def _final_tc(wv2, z2, srep, o_w, o_b, gn2_w, gn2_b, gn2_ms,
              ffn1_w, ffn1_b, ffn2_w, ffn2_b):
    return pl.pallas_call(
        _final_body,
        out_shape=jax.ShapeDtypeStruct((N, D), jnp.float32),
    )(wv2, z2, srep, o_w, o_b, gn2_w, gn2_b, gn2_ms,
      ffn1_w, ffn1_b, ffn2_w, ffn2_b)


@jax.jit
def kernel(h, edge_index, etypes, basis_q, w_comp_q, bias_q, basis_k,
           w_comp_k, bias_k, basis_v, w_comp_v, bias_v, gn1_w, gn1_b, gn1_ms,
           gn2_w, gn2_b, gn2_ms, o_w, o_b, ffn1_w, ffn1_b, ffn2_w, ffn2_b):
    src = edge_index[0]
    dst = edge_index[1]

    hn = _graph_norm_tc(h, gn1_w, gn1_b, gn1_ms)
    tq, tk, tv = _tables_tc(hn, basis_q, w_comp_q, basis_k, w_comp_k,
                            basis_v, w_comp_v)
    tq = tq.reshape(R * N, D)
    tk = tk.reshape(R * N, D)
    tv = tv.reshape(R * N, D)

    z128 = jnp.zeros((16, D), jnp.float32)
    aggq, aggk, aggv = _phase1_sc(tq, tk, tv, src, dst, etypes, z128)

    qt, kt, vt = _qkv_tc(aggq, aggk, aggv, bias_q, bias_k, bias_v)

    kg, qg = _gather_sc(kt, qt, src, dst)
    scr = _score_tc(kg, qg)
    wv2, z2 = _phase2_sc(vt, scr, src, dst, z128)

    rows = jnp.arange(D)[:, None]
    cols = jnp.arange(D)[None, :]
    srep = jnp.where((cols // DH == rows) & (rows < H), 1.0, 0.0)
    srep = srep.astype(jnp.float32)

    return _final_tc(wv2, z2, srep, o_w, o_b, gn2_w, gn2_b, gn2_ms,
                     ffn1_w, ffn1_b, ffn2_w, ffn2_b)


# speedup vs baseline: 27.5558x; 1.3050x over previous
import functools
import jax
import jax.numpy as jnp
from jax import lax
from jax.experimental import pallas as pl
from jax.experimental.pallas import tpu as pltpu
from jax.experimental.pallas import tpu_sc as plsc

N = 10000
E = 320000
D = 128
H = 8
DH = 16
R = 9

NC = 2
NS = 16
NW = NC * NS
EPT = E // NW
CH = 80
NCHUNK = EPT // CH
AGG_ROWS = N + 16
RPT = 624

_mesh = functools.partial(
    plsc.VectorSubcoreMesh,
    core_axis_name="c", subcore_axis_name="s",
    num_cores=NC, num_subcores=NS)


def _gn_body(t_ref, w_ref, b_ref, ms_ref, o_ref):
    t = t_ref[...]
    mean = jnp.mean(t, axis=0)
    sub = t - mean * ms_ref[...]
    std = jnp.sqrt(jnp.mean(sub * sub, axis=0) + 1e-06)
    o_ref[...] = w_ref[...] * sub / std + b_ref[...]


def _graph_norm_tc(t, w, b, ms):
    return pl.pallas_call(
        _gn_body,
        out_shape=jax.ShapeDtypeStruct((N, D), jnp.float32),
    )(t, w, b, ms)


def _tables_body(hn_ref, bq_ref, wq_ref, bk_ref, wk_ref, bv_ref, wv_ref,
                 oq_ref, ok_ref, ov_ref):
    hn = hn_ref[...]
    for b_ref, w_ref, o_ref in ((bq_ref, wq_ref, oq_ref),
                                (bk_ref, wk_ref, ok_ref),
                                (bv_ref, wv_ref, ov_ref)):
        W = w_ref[0, 0, 0] * b_ref[0]
        for bb in range(1, R):
            W = W + w_ref[0, 0, bb] * b_ref[bb]
        o_ref[0] = jnp.dot(hn, W, preferred_element_type=jnp.float32,
                 precision=lax.Precision.HIGHEST)


def _tables_tc(hn, basis_q, w_comp_q, basis_k, w_comp_k, basis_v, w_comp_v):
    BR = 2000
    full3 = pl.BlockSpec((R, D, D), lambda r, i: (0, 0, 0))
    hblk = pl.BlockSpec((BR, D), lambda r, i: (i, 0))
    wrow = pl.BlockSpec((1, 1, R), lambda r, i: (r, 0, 0))
    outb = pl.BlockSpec((1, BR, D), lambda r, i: (r, i, 0))
    out = jax.ShapeDtypeStruct((R, N, D), jnp.float32)
    return pl.pallas_call(
        _tables_body,
        grid=(R, N // BR),
        in_specs=[hblk, full3, wrow, full3, wrow, full3, wrow],
        out_specs=[outb, outb, outb],
        out_shape=[out, out, out],
    )(hn, basis_q, w_comp_q.reshape(R, 1, R), basis_k,
      w_comp_k.reshape(R, 1, R), basis_v, w_comp_v.reshape(R, 1, R))


def _phase1_body(tq_h, tk_h, tv_h, src_h, dst_h, et_h, z_h,
                 oq_h, ok_h, ov_h,
                 gidx2, didxs2, tmp_s, tmp_e, tmp_d, zbuf, rbuf0, rbuf1,
                 gsem0, gsem1, agg):
    cid = lax.axis_index("c")
    sid = lax.axis_index("s")
    wid = sid * NC + cid
    base = wid * EPT

    def build_row(j, slot):
        pltpu.sync_copy(src_h.at[pl.ds(base + j * CH, CH)], tmp_s)
        pltpu.sync_copy(et_h.at[pl.ds(base + j * CH, CH)], tmp_e)
        pltpu.sync_copy(dst_h.at[pl.ds(base + j * CH, CH)], tmp_d)
        for k in range(CH // 16):
            sl = pl.ds(k * 16, 16)
            gidx2[slot, sl] = tmp_e[sl] * N + tmp_s[sl]
            didxs2[slot, sl] = tmp_d[sl]

    pltpu.sync_copy(z_h.at[pl.ds(0, 8)], zbuf)

    r0 = sid * RPT
    for t_h, o_h in ((tq_h, oq_h), (tk_h, ok_h), (tv_h, ov_h)):
        def zero_body(k, carry):
            pltpu.sync_copy(zbuf, agg.at[pl.ds(r0 + k * 8, 8)])
            return carry

        lax.fori_loop(0, RPT // 8, zero_body, 0)

        @pl.when(sid == 0)
        def _():
            for k in range((AGG_ROWS - NS * RPT) // 8):
                pltpu.sync_copy(zbuf, agg.at[pl.ds(NS * RPT + k * 8, 8)])
        plsc.subcore_barrier()

        def g_issue(slot, rb, sem):
            pltpu.async_copy(t_h.at[gidx2.at[slot]], rb, sem)

        def g_wait(rb, sem):
            pltpu.make_async_copy(t_h.at[gidx2.at[0]], rb, sem).wait()

        def s_do(slot, rb):
            pltpu.sync_copy(rb, agg.at[didxs2.at[slot]], add=True)

        build_row(0, 0)
        g_issue(0, rbuf0, gsem0)

        def pair(j2, carry):
            a = 2 * j2

            @pl.when(a + 1 < NCHUNK)
            def _():
                build_row(a + 1, 1)
                g_issue(1, rbuf1, gsem1)
            g_wait(rbuf0, gsem0)
            s_do(0, rbuf0)

            @pl.when(a + 2 < NCHUNK)
            def _():
                build_row(a + 2, 0)
                g_issue(0, rbuf0, gsem0)

            @pl.when(a + 1 < NCHUNK)
            def _():
                g_wait(rbuf1, gsem1)
                s_do(1, rbuf1)
            return carry

        lax.fori_loop(0, (NCHUNK + 1) // 2, pair, 0)
        plsc.subcore_barrier()

        for k in range(RPT // CH):
            pltpu.sync_copy(agg.at[pl.ds(r0 + k * CH, CH)], rbuf0)
            pltpu.sync_copy(rbuf0, o_h.at[cid, pl.ds(r0 + k * CH, CH)])
        rem = RPT - (RPT // CH) * CH
        pltpu.sync_copy(agg.at[pl.ds(r0 + RPT - rem, rem)], rbuf0.at[pl.ds(0, rem)])
        pltpu.sync_copy(rbuf0.at[pl.ds(0, rem)],
                        o_h.at[cid, pl.ds(r0 + RPT - rem, rem)])

        @pl.when(sid == 0)
        def _():
            nrem = N - NS * RPT
            pltpu.sync_copy(agg.at[pl.ds(NS * RPT, nrem)], rbuf0.at[pl.ds(0, nrem)])
            pltpu.sync_copy(rbuf0.at[pl.ds(0, nrem)], o_h.at[cid, pl.ds(NS * RPT, nrem)])
        plsc.subcore_barrier()


def _phase1_sc(tq, tk, tv, src, dst, etypes, z128):
    out = jax.ShapeDtypeStruct((NC, N, D), jnp.float32)
    return pl.kernel(
        _phase1_body,
        out_type=[out, out, out],
        mesh=_mesh(),
        scratch_types=[
            pltpu.VMEM((2, CH), jnp.int32),
            pltpu.VMEM((2, CH), jnp.int32),
            pltpu.VMEM((CH,), jnp.int32),
            pltpu.VMEM((CH,), jnp.int32),
            pltpu.VMEM((CH,), jnp.int32),
            pltpu.VMEM((8, D), jnp.float32),
            pltpu.VMEM((CH, D), jnp.float32),
            pltpu.VMEM((CH, D), jnp.float32),
            pltpu.SemaphoreType.DMA,
            pltpu.SemaphoreType.DMA,
            pltpu.VMEM_SHARED((AGG_ROWS, D), jnp.float32),
        ],
    )(tq, tk, tv, src, dst, etypes, z128)


def _qkv_body(aq_ref, ak_ref, av_ref, bq_ref, bk_ref, bv_ref,
              qt_ref, kt_ref, vt_ref):
    qt_ref[...] = jnp.maximum(aq_ref[0] + aq_ref[1] + bq_ref[...], 0.0)
    kt_ref[...] = jnp.maximum(ak_ref[0] + ak_ref[1] + bk_ref[...], 0.0)
    vt_ref[...] = jnp.maximum(av_ref[0] + av_ref[1] + bv_ref[...], 0.0)


def _qkv_tc(aggq, aggk, aggv, bias_q, bias_k, bias_v):
    BR = 2000
    inb = pl.BlockSpec((NC, BR, D), lambda i: (0, i, 0))
    bb = pl.BlockSpec((D,), lambda i: (0,))
    ob = pl.BlockSpec((BR, D), lambda i: (i, 0))
    os = jax.ShapeDtypeStruct((N, D), jnp.float32)
    return pl.pallas_call(
        _qkv_body,
        grid=(N // BR,),
        in_specs=[inb, inb, inb, bb, bb, bb],
        out_specs=[ob, ob, ob],
        out_shape=[os, os, os],
    )(aggq, aggk, aggv, bias_q, bias_k, bias_v)


C2 = 64
NF2 = EPT // C2
TAIL2 = EPT - NF2 * C2


CG = 128
NFG = EPT // CG
TAILG = EPT - NFG * CG


def _gather_body(kt_h, qt_h, src_h, dst_h, kg_h, qg_h,
                 sidx, didxg, tmp_s, tmp_d,
                 kbuf0, qbuf0, kbuf1, qbuf1, sem0, sem1):
    cid = lax.axis_index("c")
    sid = lax.axis_index("s")
    wid = sid * NC + cid
    base = wid * EPT

    def build(j, slot, nreal):
        pltpu.sync_copy(src_h.at[pl.ds(base + j * CG, nreal)],
                        tmp_s.at[pl.ds(0, nreal)])
        pltpu.sync_copy(dst_h.at[pl.ds(base + j * CG, nreal)],
                        tmp_d.at[pl.ds(0, nreal)])
        for k in range(CG // 16):
            sl = pl.ds(k * 16, 16)
            if k < nreal // 16:
                sidx[slot, sl] = tmp_s[sl]
                didxg[slot, sl] = tmp_d[sl]
            else:
                sidx[slot, sl] = jnp.zeros((16,), jnp.int32)
                didxg[slot, sl] = jnp.zeros((16,), jnp.int32)

    def g_issue(slot, kb, qb, sem):
        pltpu.async_copy(kt_h.at[sidx.at[slot]], kb, sem)
        pltpu.async_copy(qt_h.at[didxg.at[slot]], qb, sem)

    def g_wait(kb, qb, sem):
        pltpu.make_async_copy(kt_h.at[sidx.at[0]], kb, sem).wait()
        pltpu.make_async_copy(qt_h.at[didxg.at[0]], qb, sem).wait()

    def w_out(j, kb, qb, nreal):
        pltpu.sync_copy(kb.at[pl.ds(0, nreal)],
                        kg_h.at[pl.ds(base + j * CG, nreal)])
        pltpu.sync_copy(qb.at[pl.ds(0, nreal)],
                        qg_h.at[pl.ds(base + j * CG, nreal)])

    build(0, 0, CG)
    g_issue(0, kbuf0, qbuf0, sem0)

    def pair(j2, carry):
        a = 2 * j2
        build(a + 1, 1, CG)
        g_issue(1, kbuf1, qbuf1, sem1)
        g_wait(kbuf0, qbuf0, sem0)
        w_out(a, kbuf0, qbuf0, CG)

        @pl.when(a + 2 < NFG)
        def _():
            build(a + 2, 0, CG)
            g_issue(0, kbuf0, qbuf0, sem0)
        g_wait(kbuf1, qbuf1, sem1)
        w_out(a + 1, kbuf1, qbuf1, CG)
        return carry

    lax.fori_loop(0, NFG // 2, pair, 0)
    build(NFG, 0, TAILG)
    g_issue(0, kbuf0, qbuf0, sem0)
    g_wait(kbuf0, qbuf0, sem0)
    w_out(NFG, kbuf0, qbuf0, TAILG)


def _gather_sc(kt, qt, src, dst):
    out = jax.ShapeDtypeStruct((E, D), jnp.float32)
    return pl.kernel(
        _gather_body,
        out_type=[out, out],
        mesh=_mesh(),
        scratch_types=[
            pltpu.VMEM((2, CG), jnp.int32),
            pltpu.VMEM((2, CG), jnp.int32),
            pltpu.VMEM((CG,), jnp.int32),
            pltpu.VMEM((CG,), jnp.int32),
            pltpu.VMEM((CG, D), jnp.float32),
            pltpu.VMEM((CG, D), jnp.float32),
            pltpu.VMEM((CG, D), jnp.float32),
            pltpu.VMEM((CG, D), jnp.float32),
            pltpu.SemaphoreType.DMA,
            pltpu.SemaphoreType.DMA,
        ],
    )(kt, qt, src, dst)


def _score_body(kg_ref, qg_ref, sel_ref, scr_ref):
    s = jnp.dot(kg_ref[...] * qg_ref[...], sel_ref[...],
                preferred_element_type=jnp.float32,
                 precision=lax.Precision.HIGHEST)
    scr_ref[...] = jnp.exp(jnp.clip(s * 0.25, -10.0, 10.0))


def _score_tc(kg, qg):
    BR = 4000
    sel = jnp.where(jnp.arange(D)[:, None] // DH == jnp.arange(16)[None, :],
                    1.0, 0.0).astype(jnp.float32)
    inb = pl.BlockSpec((BR, D), lambda i: (i, 0))
    return pl.pallas_call(
        _score_body,
        grid=(E // BR,),
        in_specs=[inb, inb, pl.BlockSpec((D, 16), lambda i: (0, 0))],
        out_specs=pl.BlockSpec((BR, 16), lambda i: (i, 0)),
        out_shape=jax.ShapeDtypeStruct((E, 16), jnp.float32),
    )(kg, qg, sel)


def _phase2_body(vt_h, scr_h, src_h, dst_h, z_h, wv_h, zo_h,
                 sidx, didxs, tmp_s, tmp_d,
                 zbuf, vbuf0, vbuf1, obuf, sbuf0, sbuf1, gsem0, gsem1, acc):
    cid = lax.axis_index("c")
    sid = lax.axis_index("s")
    wid = sid * NC + cid
    base = wid * EPT
    r0 = sid * RPT

    pltpu.sync_copy(z_h, zbuf)

    def zero_acc():
        def zero_body(k, carry):
            pltpu.sync_copy(zbuf, acc.at[pl.ds(r0 + k * 16, 16)])
            return carry

        lax.fori_loop(0, RPT // 16, zero_body, 0)

        @pl.when(sid == 0)
        def _():
            for k in range((AGG_ROWS - NS * RPT) // 16):
                pltpu.sync_copy(zbuf, acc.at[pl.ds(NS * RPT + k * 16, 16)])

    def writeback(o_h):
        def wb_body(k, carry):
            pltpu.sync_copy(acc.at[pl.ds(r0 + k * C2, C2)], obuf)
            pltpu.sync_copy(obuf, o_h.at[cid, pl.ds(r0 + k * C2, C2)])
            return carry

        lax.fori_loop(0, RPT // C2, wb_body, 0)
        rem = RPT - (RPT // C2) * C2
        pltpu.sync_copy(acc.at[pl.ds(r0 + RPT - rem, rem)],
                        obuf.at[pl.ds(0, rem)])
        pltpu.sync_copy(obuf.at[pl.ds(0, rem)],
                        o_h.at[cid, pl.ds(r0 + RPT - rem, rem)])

        @pl.when(sid == 0)
        def _():
            nrem = N - NS * RPT
            pltpu.sync_copy(acc.at[pl.ds(NS * RPT, nrem)],
                            obuf.at[pl.ds(0, nrem)])
            pltpu.sync_copy(obuf.at[pl.ds(0, nrem)],
                            o_h.at[cid, pl.ds(NS * RPT, nrem)])

    zero_acc()
    plsc.subcore_barrier()

    def build_a(j, slot, nreal):
        pltpu.sync_copy(src_h.at[pl.ds(base + j * C2, nreal)],
                        tmp_s.at[pl.ds(0, nreal)])
        pltpu.sync_copy(dst_h.at[pl.ds(base + j * C2, nreal)],
                        tmp_d.at[pl.ds(0, nreal)])
        for k in range(C2 // 16):
            sl = pl.ds(k * 16, 16)
            if k < nreal // 16:
                sidx[slot, sl] = tmp_s[sl]
                didxs[slot, sl] = tmp_d[sl]
            else:
                sidx[slot, sl] = jnp.zeros((16,), jnp.int32)
                didxs[slot, sl] = jnp.full((16,), N, jnp.int32)

    def issue_a(j, slot, vb, sb, sem, nreal):
        pltpu.async_copy(vt_h.at[sidx.at[slot]], vb, sem)
        pltpu.async_copy(scr_h.at[pl.ds(base + j * C2, nreal)],
                         sb.at[pl.ds(0, nreal)], sem)

    def wait_a(vb, sb, sem, nreal):
        pltpu.make_async_copy(vt_h.at[sidx.at[0]], vb, sem).wait()
        pltpu.make_async_copy(scr_h.at[pl.ds(base, nreal)],
                              sb.at[pl.ds(0, nreal)], sem).wait()

    def work_a(slot, vb, sb):
        def edge(e, c2):
            es = sbuf0[e, pl.ds(0, 16)] if sb is sbuf0 else sbuf1[e, pl.ds(0, 16)]
            for hh in range(H):
                obuf[e, pl.ds(hh * DH, DH)] = (
                    vb[e, pl.ds(hh * DH, DH)] * es[hh])
            return c2

        lax.fori_loop(0, C2, edge, 0)
        pltpu.sync_copy(obuf, acc.at[didxs.at[slot]], add=True)

    build_a(0, 0, C2)
    issue_a(0, 0, vbuf0, sbuf0, gsem0, C2)

    def pair_a(j2, carry):
        a = 2 * j2
        build_a(a + 1, 1, C2)
        issue_a(a + 1, 1, vbuf1, sbuf1, gsem1, C2)
        wait_a(vbuf0, sbuf0, gsem0, C2)
        work_a(0, vbuf0, sbuf0)

        @pl.when(a + 2 < NF2)
        def _():
            build_a(a + 2, 0, C2)
            issue_a(a + 2, 0, vbuf0, sbuf0, gsem0, C2)
        wait_a(vbuf1, sbuf1, gsem1, C2)
        work_a(1, vbuf1, sbuf1)
        return carry

    lax.fori_loop(0, NF2 // 2, pair_a, 0)
    build_a(NF2, 0, TAIL2)
    issue_a(NF2, 0, vbuf0, sbuf0, gsem0, TAIL2)
    wait_a(vbuf0, sbuf0, gsem0, TAIL2)
    work_a(0, vbuf0, sbuf0)
    plsc.subcore_barrier()
    writeback(wv_h)
    plsc.subcore_barrier()

    zero_acc()

    def zrow_init(e, carry):
        for k in range(1, D // 16):
            obuf[e, pl.ds(k * 16, 16)] = jnp.zeros((16,), jnp.float32)
        return carry

    lax.fori_loop(0, C2, zrow_init, 0)
    plsc.subcore_barrier()

    def issue_b(j, slot, sb, sem, nreal):
        pltpu.sync_copy(dst_h.at[pl.ds(base + j * C2, nreal)],
                        tmp_d.at[pl.ds(0, nreal)])
        for k in range(C2 // 16):
            sl = pl.ds(k * 16, 16)
            if k < nreal // 16:
                didxs[slot, sl] = tmp_d[sl]
            else:
                didxs[slot, sl] = jnp.full((16,), N, jnp.int32)
        pltpu.async_copy(scr_h.at[pl.ds(base + j * C2, nreal)],
                         sb.at[pl.ds(0, nreal)], sem)

    def wait_b(sb, sem, nreal):
        pltpu.make_async_copy(scr_h.at[pl.ds(base, nreal)],
                              sb.at[pl.ds(0, nreal)], sem).wait()

    def work_b(slot, sb):
        def zrow(e, c2):
            obuf[e, pl.ds(0, 16)] = (
                sbuf0[e, pl.ds(0, 16)] if sb is sbuf0
                else sbuf1[e, pl.ds(0, 16)])
            return c2

        lax.fori_loop(0, C2, zrow, 0)
        pltpu.sync_copy(obuf, acc.at[didxs.at[slot]], add=True)

    issue_b(0, 0, sbuf0, gsem0, C2)

    def pair_b(j2, carry):
        a = 2 * j2
        issue_b(a + 1, 1, sbuf1, gsem1, C2)
        wait_b(sbuf0, gsem0, C2)
        work_b(0, sbuf0)

        @pl.when(a + 2 < NF2)
        def _():
            issue_b(a + 2, 0, sbuf0, gsem0, C2)
        wait_b(sbuf1, gsem1, C2)
        work_b(1, sbuf1)
        return carry

    lax.fori_loop(0, NF2 // 2, pair_b, 0)
    issue_b(NF2, 0, sbuf0, gsem0, TAIL2)
    wait_b(sbuf0, gsem0, TAIL2)
    work_b(0, sbuf0)
    plsc.subcore_barrier()
    writeback(zo_h)


def _phase2_sc(vt, scr, src, dst, z128):
    out = jax.ShapeDtypeStruct((NC, N, D), jnp.float32)
    return pl.kernel(
        _phase2_body,
        out_type=[out, out],
        mesh=_mesh(),
        scratch_types=[
            pltpu.VMEM((2, C2), jnp.int32),
            pltpu.VMEM((2, C2), jnp.int32),
            pltpu.VMEM((C2,), jnp.int32),
            pltpu.VMEM((C2,), jnp.int32),
            pltpu.VMEM((16, D), jnp.float32),
            pltpu.VMEM((C2, D), jnp.float32),
            pltpu.VMEM((C2, D), jnp.float32),
            pltpu.VMEM((C2, D), jnp.float32),
            pltpu.VMEM((C2, 16), jnp.float32),
            pltpu.VMEM((C2, 16), jnp.float32),
            pltpu.SemaphoreType.DMA,
            pltpu.SemaphoreType.DMA,
            pltpu.VMEM_SHARED((AGG_ROWS, D), jnp.float32),
        ],
    )(vt, scr, src, dst, z128)


def _final_body(wv_ref, zo_ref, srep_ref, ow_ref, ob_ref, gw_ref, gb_ref,
                gms_ref, f1w_ref, f1b_ref, f2w_ref, f2b_ref, o_ref):
    wv = wv_ref[0] + wv_ref[1]
    z = zo_ref[0] + zo_ref[1]
    zrep = jnp.dot(z, srep_ref[...], preferred_element_type=jnp.float32,
                 precision=lax.Precision.HIGHEST)
    attn = wv / (zrep + 1e-06)
    h2 = jnp.dot(attn, ow_ref[...], preferred_element_type=jnp.float32,
                 precision=lax.Precision.HIGHEST) + ob_ref[...]
    mean = jnp.mean(h2, axis=0)
    sub = h2 - mean * gms_ref[...]
    std = jnp.sqrt(jnp.mean(sub * sub, axis=0) + 1e-06)
    h2n = gw_ref[...] * sub / std + gb_ref[...]
    ff = jnp.maximum(
        jnp.dot(h2n, f1w_ref[...], preferred_element_type=jnp.float32,
                 precision=lax.Precision.HIGHEST)
        + f1b_ref[...], 0.0)
    o_ref[...] = jnp.dot(ff, f2w_ref[...],
                         preferred_element_type=jnp.float32,
                 precision=lax.Precision.HIGHEST) + f2b_ref[...]


def _final_tc(wv2, z2, srep, o_w, o_b, gn2_w, gn2_b, gn2_ms,
              ffn1_w, ffn1_b, ffn2_w, ffn2_b):
    return pl.pallas_call(
        _final_body,
        out_shape=jax.ShapeDtypeStruct((N, D), jnp.float32),
    )(wv2, z2, srep, o_w, o_b, gn2_w, gn2_b, gn2_ms,
      ffn1_w, ffn1_b, ffn2_w, ffn2_b)


@jax.jit
def kernel(h, edge_index, etypes, basis_q, w_comp_q, bias_q, basis_k,
           w_comp_k, bias_k, basis_v, w_comp_v, bias_v, gn1_w, gn1_b, gn1_ms,
           gn2_w, gn2_b, gn2_ms, o_w, o_b, ffn1_w, ffn1_b, ffn2_w, ffn2_b):
    src = edge_index[0]
    dst = edge_index[1]

    hn = _graph_norm_tc(h, gn1_w, gn1_b, gn1_ms)
    tq, tk, tv = _tables_tc(hn, basis_q, w_comp_q, basis_k, w_comp_k,
                            basis_v, w_comp_v)
    tq = tq.reshape(R * N, D)
    tk = tk.reshape(R * N, D)
    tv = tv.reshape(R * N, D)

    z128 = jnp.zeros((16, D), jnp.float32)
    aggq, aggk, aggv = _phase1_sc(tq, tk, tv, src, dst, etypes, z128)

    qt, kt, vt = _qkv_tc(aggq, aggk, aggv, bias_q, bias_k, bias_v)

    kg, qg = _gather_sc(kt, qt, src, dst)
    scr = _score_tc(kg, qg)
    wv2, z2 = _phase2_sc(vt, scr, src, dst, z128)

    rows = jnp.arange(D)[:, None]
    cols = jnp.arange(D)[None, :]
    srep = jnp.where((cols // DH == rows) & (rows < H), 1.0, 0.0)
    srep = srep.astype(jnp.float32)

    return _final_tc(wv2, z2, srep, o_w, o_b, gn2_w, gn2_b, gn2_ms,
                     ffn1_w, ffn1_b, ffn2_w, ffn2_b)


# speedup vs baseline: 29.6730x; 1.0768x over previous
import functools
import jax
import jax.numpy as jnp
from jax import lax
from jax.experimental import pallas as pl
from jax.experimental.pallas import tpu as pltpu
from jax.experimental.pallas import tpu_sc as plsc

N = 10000
E = 320000
D = 128
H = 8
DH = 16
R = 9

NC = 2
NS = 16
NW = NC * NS
EPT = E // NW
CH = 80
NCHUNK = EPT // CH
AGG_ROWS = N + 16
RPT = 624

_mesh = functools.partial(
    plsc.VectorSubcoreMesh,
    core_axis_name="c", subcore_axis_name="s",
    num_cores=NC, num_subcores=NS)


def _gn_body(t_ref, w_ref, b_ref, ms_ref, o_ref):
    t = t_ref[...]
    mean = jnp.mean(t, axis=0)
    sub = t - mean * ms_ref[...]
    std = jnp.sqrt(jnp.mean(sub * sub, axis=0) + 1e-06)
    o_ref[...] = w_ref[...] * sub / std + b_ref[...]


def _graph_norm_tc(t, w, b, ms):
    return pl.pallas_call(
        _gn_body,
        out_shape=jax.ShapeDtypeStruct((N, D), jnp.float32),
    )(t, w, b, ms)


def _tables_body(hn_ref, bq_ref, wq_ref, bk_ref, wk_ref, bv_ref, wv_ref,
                 oq_ref, ok_ref, ov_ref):
    hn = hn_ref[...]
    for b_ref, w_ref, o_ref in ((bq_ref, wq_ref, oq_ref),
                                (bk_ref, wk_ref, ok_ref),
                                (bv_ref, wv_ref, ov_ref)):
        W = w_ref[0, 0, 0] * b_ref[0]
        for bb in range(1, R):
            W = W + w_ref[0, 0, bb] * b_ref[bb]
        o_ref[0] = jnp.dot(hn, W, preferred_element_type=jnp.float32,
                 precision=lax.Precision.HIGHEST)


def _tables_tc(hn, basis_q, w_comp_q, basis_k, w_comp_k, basis_v, w_comp_v):
    BR = 2000
    full3 = pl.BlockSpec((R, D, D), lambda r, i: (0, 0, 0))
    hblk = pl.BlockSpec((BR, D), lambda r, i: (i, 0))
    wrow = pl.BlockSpec((1, 1, R), lambda r, i: (r, 0, 0))
    outb = pl.BlockSpec((1, BR, D), lambda r, i: (r, i, 0))
    out = jax.ShapeDtypeStruct((R, N, D), jnp.float32)
    return pl.pallas_call(
        _tables_body,
        grid=(R, N // BR),
        in_specs=[hblk, full3, wrow, full3, wrow, full3, wrow],
        out_specs=[outb, outb, outb],
        out_shape=[out, out, out],
    )(hn, basis_q, w_comp_q.reshape(R, 1, R), basis_k,
      w_comp_k.reshape(R, 1, R), basis_v, w_comp_v.reshape(R, 1, R))


def _gidx_body(et_ref, src_ref, o_ref):
    o_ref[...] = et_ref[...] * N + src_ref[...]


def _gidx_tc(etypes, src):
    return pl.pallas_call(
        _gidx_body,
        out_shape=jax.ShapeDtypeStruct((E // 128, 128), jnp.int32),
    )(etypes.reshape(E // 128, 128), src.reshape(E // 128, 128)).reshape(E)


def _phase1_body(tq_h, tk_h, tv_h, gix_h, dst_h, z_h,
                 oq_h, ok_h, ov_h,
                 gidx2, didxs2, zbuf, rbuf0, rbuf1,
                 gsem0, gsem1, agg):
    cid = lax.axis_index("c")
    sid = lax.axis_index("s")
    wid = sid * NC + cid
    base = wid * EPT

    def build_row(j, slot):
        pltpu.sync_copy(gix_h.at[pl.ds(base + j * CH, CH)], gidx2.at[slot])
        pltpu.sync_copy(dst_h.at[pl.ds(base + j * CH, CH)], didxs2.at[slot])

    pltpu.sync_copy(z_h.at[pl.ds(0, 8)], zbuf)

    r0 = sid * RPT
    for t_h, o_h in ((tq_h, oq_h), (tk_h, ok_h), (tv_h, ov_h)):
        def zero_body(k, carry):
            pltpu.sync_copy(zbuf, agg.at[pl.ds(r0 + k * 8, 8)])
            return carry

        lax.fori_loop(0, RPT // 8, zero_body, 0)

        @pl.when(sid == 0)
        def _():
            for k in range((AGG_ROWS - NS * RPT) // 8):
                pltpu.sync_copy(zbuf, agg.at[pl.ds(NS * RPT + k * 8, 8)])
        plsc.subcore_barrier()

        def g_issue(slot, rb, sem):
            pltpu.async_copy(t_h.at[gidx2.at[slot]], rb, sem)

        def g_wait(rb, sem):
            pltpu.make_async_copy(t_h.at[gidx2.at[0]], rb, sem).wait()

        def s_do(slot, rb):
            pltpu.sync_copy(rb, agg.at[didxs2.at[slot]], add=True)

        build_row(0, 0)
        g_issue(0, rbuf0, gsem0)

        def pair(j2, carry):
            a = 2 * j2

            @pl.when(a + 1 < NCHUNK)
            def _():
                build_row(a + 1, 1)
                g_issue(1, rbuf1, gsem1)
            g_wait(rbuf0, gsem0)
            s_do(0, rbuf0)

            @pl.when(a + 2 < NCHUNK)
            def _():
                build_row(a + 2, 0)
                g_issue(0, rbuf0, gsem0)

            @pl.when(a + 1 < NCHUNK)
            def _():
                g_wait(rbuf1, gsem1)
                s_do(1, rbuf1)
            return carry

        lax.fori_loop(0, (NCHUNK + 1) // 2, pair, 0)
        plsc.subcore_barrier()

        for k in range(RPT // CH):
            pltpu.sync_copy(agg.at[pl.ds(r0 + k * CH, CH)], rbuf0)
            pltpu.sync_copy(rbuf0, o_h.at[cid, pl.ds(r0 + k * CH, CH)])
        rem = RPT - (RPT // CH) * CH
        pltpu.sync_copy(agg.at[pl.ds(r0 + RPT - rem, rem)], rbuf0.at[pl.ds(0, rem)])
        pltpu.sync_copy(rbuf0.at[pl.ds(0, rem)],
                        o_h.at[cid, pl.ds(r0 + RPT - rem, rem)])

        @pl.when(sid == 0)
        def _():
            nrem = N - NS * RPT
            pltpu.sync_copy(agg.at[pl.ds(NS * RPT, nrem)], rbuf0.at[pl.ds(0, nrem)])
            pltpu.sync_copy(rbuf0.at[pl.ds(0, nrem)], o_h.at[cid, pl.ds(NS * RPT, nrem)])
        plsc.subcore_barrier()


def _phase1_sc(tq, tk, tv, gix, dst, z128):
    out = jax.ShapeDtypeStruct((NC, N, D), jnp.float32)
    return pl.kernel(
        _phase1_body,
        out_type=[out, out, out],
        mesh=_mesh(),
        scratch_types=[
            pltpu.VMEM((2, CH), jnp.int32),
            pltpu.VMEM((2, CH), jnp.int32),
            pltpu.VMEM((8, D), jnp.float32),
            pltpu.VMEM((CH, D), jnp.float32),
            pltpu.VMEM((CH, D), jnp.float32),
            pltpu.SemaphoreType.DMA,
            pltpu.SemaphoreType.DMA,
            pltpu.VMEM_SHARED((AGG_ROWS, D), jnp.float32),
        ],
    )(tq, tk, tv, gix, dst, z128)


def _qkv_body(aq_ref, ak_ref, av_ref, bq_ref, bk_ref, bv_ref,
              qt_ref, kt_ref, vt_ref):
    qt_ref[...] = jnp.maximum(aq_ref[0] + aq_ref[1] + bq_ref[...], 0.0)
    kt_ref[...] = jnp.maximum(ak_ref[0] + ak_ref[1] + bk_ref[...], 0.0)
    vt_ref[...] = jnp.maximum(av_ref[0] + av_ref[1] + bv_ref[...], 0.0)


def _qkv_tc(aggq, aggk, aggv, bias_q, bias_k, bias_v):
    BR = 2000
    inb = pl.BlockSpec((NC, BR, D), lambda i: (0, i, 0))
    bb = pl.BlockSpec((D,), lambda i: (0,))
    ob = pl.BlockSpec((BR, D), lambda i: (i, 0))
    os = jax.ShapeDtypeStruct((N, D), jnp.float32)
    return pl.pallas_call(
        _qkv_body,
        grid=(N // BR,),
        in_specs=[inb, inb, inb, bb, bb, bb],
        out_specs=[ob, ob, ob],
        out_shape=[os, os, os],
    )(aggq, aggk, aggv, bias_q, bias_k, bias_v)


C2 = 64
NF2 = EPT // C2
TAIL2 = EPT - NF2 * C2


CG = 128
NFG = EPT // CG
TAILG = EPT - NFG * CG


def _gather_body(kt_h, qt_h, src_h, dst_h, kg_h, qg_h,
                 sidx, didxg, tmp_s, tmp_d,
                 kbuf0, qbuf0, kbuf1, qbuf1, sem0, sem1):
    cid = lax.axis_index("c")
    sid = lax.axis_index("s")
    wid = sid * NC + cid
    base = wid * EPT

    def build(j, slot, nreal):
        pltpu.sync_copy(src_h.at[pl.ds(base + j * CG, nreal)],
                        tmp_s.at[pl.ds(0, nreal)])
        pltpu.sync_copy(dst_h.at[pl.ds(base + j * CG, nreal)],
                        tmp_d.at[pl.ds(0, nreal)])
        for k in range(CG // 16):
            sl = pl.ds(k * 16, 16)
            if k < nreal // 16:
                sidx[slot, sl] = tmp_s[sl]
                didxg[slot, sl] = tmp_d[sl]
            else:
                sidx[slot, sl] = jnp.zeros((16,), jnp.int32)
                didxg[slot, sl] = jnp.zeros((16,), jnp.int32)

    def g_issue(slot, kb, qb, sem):
        pltpu.async_copy(kt_h.at[sidx.at[slot]], kb, sem)
        pltpu.async_copy(qt_h.at[didxg.at[slot]], qb, sem)

    def g_wait(kb, qb, sem):
        pltpu.make_async_copy(kt_h.at[sidx.at[0]], kb, sem).wait()
        pltpu.make_async_copy(qt_h.at[didxg.at[0]], qb, sem).wait()

    def w_out(j, kb, qb, nreal):
        pltpu.sync_copy(kb.at[pl.ds(0, nreal)],
                        kg_h.at[pl.ds(base + j * CG, nreal)])
        pltpu.sync_copy(qb.at[pl.ds(0, nreal)],
                        qg_h.at[pl.ds(base + j * CG, nreal)])

    build(0, 0, CG)
    g_issue(0, kbuf0, qbuf0, sem0)

    def pair(j2, carry):
        a = 2 * j2
        build(a + 1, 1, CG)
        g_issue(1, kbuf1, qbuf1, sem1)
        g_wait(kbuf0, qbuf0, sem0)
        w_out(a, kbuf0, qbuf0, CG)

        @pl.when(a + 2 < NFG)
        def _():
            build(a + 2, 0, CG)
            g_issue(0, kbuf0, qbuf0, sem0)
        g_wait(kbuf1, qbuf1, sem1)
        w_out(a + 1, kbuf1, qbuf1, CG)
        return carry

    lax.fori_loop(0, NFG // 2, pair, 0)
    build(NFG, 0, TAILG)
    g_issue(0, kbuf0, qbuf0, sem0)
    g_wait(kbuf0, qbuf0, sem0)
    w_out(NFG, kbuf0, qbuf0, TAILG)


def _gather_sc(kt, qt, src, dst):
    out = jax.ShapeDtypeStruct((E, D), jnp.float32)
    return pl.kernel(
        _gather_body,
        out_type=[out, out],
        mesh=_mesh(),
        scratch_types=[
            pltpu.VMEM((2, CG), jnp.int32),
            pltpu.VMEM((2, CG), jnp.int32),
            pltpu.VMEM((CG,), jnp.int32),
            pltpu.VMEM((CG,), jnp.int32),
            pltpu.VMEM((CG, D), jnp.float32),
            pltpu.VMEM((CG, D), jnp.float32),
            pltpu.VMEM((CG, D), jnp.float32),
            pltpu.VMEM((CG, D), jnp.float32),
            pltpu.SemaphoreType.DMA,
            pltpu.SemaphoreType.DMA,
        ],
    )(kt, qt, src, dst)


def _score_body(kg_ref, qg_ref, sel_ref, scr_ref):
    s = jnp.dot(kg_ref[...] * qg_ref[...], sel_ref[...],
                preferred_element_type=jnp.float32,
                 precision=lax.Precision.HIGHEST)
    scr_ref[...] = jnp.exp(jnp.clip(s * 0.25, -10.0, 10.0))


def _score_tc(kg, qg):
    BR = 4000
    sel = jnp.where(jnp.arange(D)[:, None] // DH == jnp.arange(16)[None, :],
                    1.0, 0.0).astype(jnp.float32)
    inb = pl.BlockSpec((BR, D), lambda i: (i, 0))
    return pl.pallas_call(
        _score_body,
        grid=(E // BR,),
        in_specs=[inb, inb, pl.BlockSpec((D, 16), lambda i: (0, 0))],
        out_specs=pl.BlockSpec((BR, 16), lambda i: (i, 0)),
        out_shape=jax.ShapeDtypeStruct((E, 16), jnp.float32),
    )(kg, qg, sel)


def _phase2_body(vt_h, scr_h, src_h, dst_h, z_h, wv_h, zo_h,
                 sidx, didxs, tmp_s, tmp_d,
                 zbuf, vbuf0, vbuf1, obuf, sbuf0, sbuf1, gsem0, gsem1, acc):
    cid = lax.axis_index("c")
    sid = lax.axis_index("s")
    wid = sid * NC + cid
    base = wid * EPT
    r0 = sid * RPT

    pltpu.sync_copy(z_h, zbuf)

    def zero_acc():
        def zero_body(k, carry):
            pltpu.sync_copy(zbuf, acc.at[pl.ds(r0 + k * 16, 16)])
            return carry

        lax.fori_loop(0, RPT // 16, zero_body, 0)

        @pl.when(sid == 0)
        def _():
            for k in range((AGG_ROWS - NS * RPT) // 16):
                pltpu.sync_copy(zbuf, acc.at[pl.ds(NS * RPT + k * 16, 16)])

    def writeback(o_h):
        def wb_body(k, carry):
            pltpu.sync_copy(acc.at[pl.ds(r0 + k * C2, C2)], obuf)
            pltpu.sync_copy(obuf, o_h.at[cid, pl.ds(r0 + k * C2, C2)])
            return carry

        lax.fori_loop(0, RPT // C2, wb_body, 0)
        rem = RPT - (RPT // C2) * C2
        pltpu.sync_copy(acc.at[pl.ds(r0 + RPT - rem, rem)],
                        obuf.at[pl.ds(0, rem)])
        pltpu.sync_copy(obuf.at[pl.ds(0, rem)],
                        o_h.at[cid, pl.ds(r0 + RPT - rem, rem)])

        @pl.when(sid == 0)
        def _():
            nrem = N - NS * RPT
            pltpu.sync_copy(acc.at[pl.ds(NS * RPT, nrem)],
                            obuf.at[pl.ds(0, nrem)])
            pltpu.sync_copy(obuf.at[pl.ds(0, nrem)],
                            o_h.at[cid, pl.ds(NS * RPT, nrem)])

    zero_acc()
    plsc.subcore_barrier()

    def build_a(j, slot, nreal):
        pltpu.sync_copy(src_h.at[pl.ds(base + j * C2, nreal)],
                        tmp_s.at[pl.ds(0, nreal)])
        pltpu.sync_copy(dst_h.at[pl.ds(base + j * C2, nreal)],
                        tmp_d.at[pl.ds(0, nreal)])
        for k in range(C2 // 16):
            sl = pl.ds(k * 16, 16)
            if k < nreal // 16:
                sidx[slot, sl] = tmp_s[sl]
                didxs[slot, sl] = tmp_d[sl]
            else:
                sidx[slot, sl] = jnp.zeros((16,), jnp.int32)
                didxs[slot, sl] = jnp.full((16,), N, jnp.int32)

    def issue_a(j, slot, vb, sb, sem, nreal):
        pltpu.async_copy(vt_h.at[sidx.at[slot]], vb, sem)
        pltpu.async_copy(scr_h.at[pl.ds(base + j * C2, nreal)],
                         sb.at[pl.ds(0, nreal)], sem)

    def wait_a(vb, sb, sem, nreal):
        pltpu.make_async_copy(vt_h.at[sidx.at[0]], vb, sem).wait()
        pltpu.make_async_copy(scr_h.at[pl.ds(base, nreal)],
                              sb.at[pl.ds(0, nreal)], sem).wait()

    def work_a(slot, vb, sb):
        def edge(e, c2):
            es = sbuf0[e, pl.ds(0, 16)] if sb is sbuf0 else sbuf1[e, pl.ds(0, 16)]
            for hh in range(H):
                obuf[e, pl.ds(hh * DH, DH)] = (
                    vb[e, pl.ds(hh * DH, DH)] * es[hh])
            return c2

        lax.fori_loop(0, C2, edge, 0)
        pltpu.sync_copy(obuf, acc.at[didxs.at[slot]], add=True)

    build_a(0, 0, C2)
    issue_a(0, 0, vbuf0, sbuf0, gsem0, C2)

    def pair_a(j2, carry):
        a = 2 * j2
        build_a(a + 1, 1, C2)
        issue_a(a + 1, 1, vbuf1, sbuf1, gsem1, C2)
        wait_a(vbuf0, sbuf0, gsem0, C2)
        work_a(0, vbuf0, sbuf0)

        @pl.when(a + 2 < NF2)
        def _():
            build_a(a + 2, 0, C2)
            issue_a(a + 2, 0, vbuf0, sbuf0, gsem0, C2)
        wait_a(vbuf1, sbuf1, gsem1, C2)
        work_a(1, vbuf1, sbuf1)
        return carry

    lax.fori_loop(0, NF2 // 2, pair_a, 0)
    build_a(NF2, 0, TAIL2)
    issue_a(NF2, 0, vbuf0, sbuf0, gsem0, TAIL2)
    wait_a(vbuf0, sbuf0, gsem0, TAIL2)
    work_a(0, vbuf0, sbuf0)
    plsc.subcore_barrier()
    writeback(wv_h)
    plsc.subcore_barrier()

    zero_acc()

    def zrow_init(e, carry):
        for k in range(1, D // 16):
            obuf[e, pl.ds(k * 16, 16)] = jnp.zeros((16,), jnp.float32)
        return carry

    lax.fori_loop(0, C2, zrow_init, 0)
    plsc.subcore_barrier()

    def issue_b(j, slot, sb, sem, nreal):
        pltpu.sync_copy(dst_h.at[pl.ds(base + j * C2, nreal)],
                        tmp_d.at[pl.ds(0, nreal)])
        for k in range(C2 // 16):
            sl = pl.ds(k * 16, 16)
            if k < nreal // 16:
                didxs[slot, sl] = tmp_d[sl]
            else:
                didxs[slot, sl] = jnp.full((16,), N, jnp.int32)
        pltpu.async_copy(scr_h.at[pl.ds(base + j * C2, nreal)],
                         sb.at[pl.ds(0, nreal)], sem)

    def wait_b(sb, sem, nreal):
        pltpu.make_async_copy(scr_h.at[pl.ds(base, nreal)],
                              sb.at[pl.ds(0, nreal)], sem).wait()

    def work_b(slot, sb):
        def zrow(e, c2):
            obuf[e, pl.ds(0, 16)] = (
                sbuf0[e, pl.ds(0, 16)] if sb is sbuf0
                else sbuf1[e, pl.ds(0, 16)])
            return c2

        lax.fori_loop(0, C2, zrow, 0)
        pltpu.sync_copy(obuf, acc.at[didxs.at[slot]], add=True)

    issue_b(0, 0, sbuf0, gsem0, C2)

    def pair_b(j2, carry):
        a = 2 * j2
        issue_b(a + 1, 1, sbuf1, gsem1, C2)
        wait_b(sbuf0, gsem0, C2)
        work_b(0, sbuf0)

        @pl.when(a + 2 < NF2)
        def _():
            issue_b(a + 2, 0, sbuf0, gsem0, C2)
        wait_b(sbuf1, gsem1, C2)
        work_b(1, sbuf1)
        return carry

    lax.fori_loop(0, NF2 // 2, pair_b, 0)
    issue_b(NF2, 0, sbuf0, gsem0, TAIL2)
    wait_b(sbuf0, gsem0, TAIL2)
    work_b(0, sbuf0)
    plsc.subcore_barrier()
    writeback(zo_h)


def _phase2_sc(vt, scr, src, dst, z128):
    out = jax.ShapeDtypeStruct((NC, N, D), jnp.float32)
    return pl.kernel(
        _phase2_body,
        out_type=[out, out],
        mesh=_mesh(),
        scratch_types=[
            pltpu.VMEM((2, C2), jnp.int32),
            pltpu.VMEM((2, C2), jnp.int32),
            pltpu.VMEM((C2,), jnp.int32),
            pltpu.VMEM((C2,), jnp.int32),
            pltpu.VMEM((16, D), jnp.float32),
            pltpu.VMEM((C2, D), jnp.float32),
            pltpu.VMEM((C2, D), jnp.float32),
            pltpu.VMEM((C2, D), jnp.float32),
            pltpu.VMEM((C2, 16), jnp.float32),
            pltpu.VMEM((C2, 16), jnp.float32),
            pltpu.SemaphoreType.DMA,
            pltpu.SemaphoreType.DMA,
            pltpu.VMEM_SHARED((AGG_ROWS, D), jnp.float32),
        ],
    )(vt, scr, src, dst, z128)


def _final_body(wv_ref, zo_ref, srep_ref, ow_ref, ob_ref, gw_ref, gb_ref,
                gms_ref, f1w_ref, f1b_ref, f2w_ref, f2b_ref, o_ref):
    wv = wv_ref[0] + wv_ref[1]
    z = zo_ref[0] + zo_ref[1]
    zrep = jnp.dot(z, srep_ref[...], preferred_element_type=jnp.float32,
                 precision=lax.Precision.HIGHEST)
    attn = wv / (zrep + 1e-06)
    h2 = jnp.dot(attn, ow_ref[...], preferred_element_type=jnp.float32,
                 precision=lax.Precision.HIGHEST) + ob_ref[...]
    mean = jnp.mean(h2, axis=0)
    sub = h2 - mean * gms_ref[...]
    std = jnp.sqrt(jnp.mean(sub * sub, axis=0) + 1e-06)
    h2n = gw_ref[...] * sub / std + gb_ref[...]
    ff = jnp.maximum(
        jnp.dot(h2n, f1w_ref[...], preferred_element_type=jnp.float32,
                 precision=lax.Precision.HIGHEST)
        + f1b_ref[...], 0.0)
    o_ref[...] = jnp.dot(ff, f2w_ref[...],
                         preferred_element_type=jnp.float32,
                 precision=lax.Precision.HIGHEST) + f2b_ref[...]


def _final_tc(wv2, z2, srep, o_w, o_b, gn2_w, gn2_b, gn2_ms,
              ffn1_w, ffn1_b, ffn2_w, ffn2_b):
    return pl.pallas_call(
        _final_body,
        out_shape=jax.ShapeDtypeStruct((N, D), jnp.float32),
    )(wv2, z2, srep, o_w, o_b, gn2_w, gn2_b, gn2_ms,
      ffn1_w, ffn1_b, ffn2_w, ffn2_b)


@jax.jit
def kernel(h, edge_index, etypes, basis_q, w_comp_q, bias_q, basis_k,
           w_comp_k, bias_k, basis_v, w_comp_v, bias_v, gn1_w, gn1_b, gn1_ms,
           gn2_w, gn2_b, gn2_ms, o_w, o_b, ffn1_w, ffn1_b, ffn2_w, ffn2_b):
    src = edge_index[0]
    dst = edge_index[1]

    hn = _graph_norm_tc(h, gn1_w, gn1_b, gn1_ms)
    tq, tk, tv = _tables_tc(hn, basis_q, w_comp_q, basis_k, w_comp_k,
                            basis_v, w_comp_v)
    tq = tq.reshape(R * N, D)
    tk = tk.reshape(R * N, D)
    tv = tv.reshape(R * N, D)

    z128 = jnp.zeros((16, D), jnp.float32)
    gix = _gidx_tc(etypes, src)
    aggq, aggk, aggv = _phase1_sc(tq, tk, tv, gix, dst, z128)

    qt, kt, vt = _qkv_tc(aggq, aggk, aggv, bias_q, bias_k, bias_v)

    kg, qg = _gather_sc(kt, qt, src, dst)
    scr = _score_tc(kg, qg)
    wv2, z2 = _phase2_sc(vt, scr, src, dst, z128)

    rows = jnp.arange(D)[:, None]
    cols = jnp.arange(D)[None, :]
    srep = jnp.where((cols // DH == rows) & (rows < H), 1.0, 0.0)
    srep = srep.astype(jnp.float32)

    return _final_tc(wv2, z2, srep, o_w, o_b, gn2_w, gn2_b, gn2_ms,
                     ffn1_w, ffn1_b, ffn2_w, ffn2_b)
